# baseline (speedup 1.0000x reference)
import functools

import jax
import jax.numpy as jnp
from jax import lax
from jax.experimental import pallas as pl
from jax.experimental.pallas import tpu as pltpu
from jax.experimental.pallas import tpu_sc as plsc

N = 10000
E = 320000
D = 128
H = 128
C = 10
G = 128

NP = 10240
NW = 32
ROWS_S = NP // 16
K = 128
NCHUNK = E // K
CH_W = NCHUNK // NW
TAIL = NCHUNK - CH_W * NW

BN = 1024
NB = NP // BN

_PREC = lax.Precision.HIGHEST


def _fill_f32(ref, n, val):
    def body(i, c):
        ref[pl.ds(i * 16, 16)] = jnp.full((16,), val, jnp.float32)
        return c
    lax.fori_loop(0, n // 16, body, 0)


def _fill2d_f32(ref, rows, val):
    def body(k, c):
        ref[k // 8, pl.ds((k % 8) * 16, 16)] = jnp.full((16,), val, jnp.float32)
        return c
    lax.fori_loop(0, rows * 8, body, 0)


def _deg_body(edge_hbm, out_hbm, acc, ones_v, zv, idx0, idx1, i0, i1):
    c = lax.axis_index("c")
    s = lax.axis_index("s")
    w = c * 16 + s
    idx = (idx0, idx1)
    isem = (i0, i1)
    _fill_f32(ones_v, K, 1.0)
    _fill_f32(zv, ROWS_S, 0.0)
    pltpu.sync_copy(zv, acc.at[pl.ds(s * ROWS_S, ROWS_S)])
    plsc.subcore_barrier()
    nck = CH_W + jnp.where(w < TAIL, 1, 0)

    def start_idx(j, b):
        off = (j * NW + w) * K
        pltpu.async_copy(edge_hbm.at[1, pl.ds(off, K)], idx[b], isem[b])

    def wait_idx(b):
        pltpu.make_async_copy(edge_hbm.at[1, pl.ds(0, K)], idx[b],
                              isem[b]).wait()

    start_idx(0, 0)
    start_idx(1, 1)

    def pair(p, carry):
        j0 = 2 * p
        wait_idx(0)
        pltpu.sync_copy(ones_v, acc.at[idx[0]], add=True)

        @pl.when(j0 + 2 < nck)
        def _():
            start_idx(j0 + 2, 0)

        wait_idx(1)
        pltpu.sync_copy(ones_v, acc.at[idx[1]], add=True)

        @pl.when(j0 + 3 < nck)
        def _():
            start_idx(j0 + 3, 1)

        return carry

    lax.fori_loop(0, CH_W // 2, pair, 0)

    @pl.when(w < TAIL)
    def _():
        wait_idx(0)
        pltpu.sync_copy(ones_v, acc.at[idx[0]], add=True)

    plsc.subcore_barrier()
    pltpu.sync_copy(acc.at[pl.ds(s * ROWS_S, ROWS_S)],
                    out_hbm.at[c, pl.ds(s * ROWS_S, ROWS_S)])


_deg_call = pl.kernel(
    _deg_body,
    out_type=jax.ShapeDtypeStruct((2, NP), jnp.float32),
    mesh=plsc.VectorSubcoreMesh(core_axis_name="c", subcore_axis_name="s"),
    scratch_types=[
        pltpu.VMEM_SHARED((NP,), jnp.float32),
        pltpu.VMEM((K,), jnp.float32),
        pltpu.VMEM((ROWS_S,), jnp.float32),
        pltpu.VMEM((K,), jnp.int32),
        pltpu.VMEM((K,), jnp.int32),
        pltpu.SemaphoreType.DMA,
        pltpu.SemaphoreType.DMA,
    ],
)


def _agg_body(y_hbm, edge_hbm, out_hbm, acc,
              sidx0, sidx1, didx0, didx1, rows0, rows1,
              is0, is1, id0, id1, gs0, gs1, ss0, ss1):
    c = lax.axis_index("c")
    s = lax.axis_index("s")
    w = c * 16 + s
    sidx = (sidx0, sidx1)
    didx = (didx0, didx1)
    rows = (rows0, rows1)
    isem = (is0, is1)
    dsem = (id0, id1)
    gsem = (gs0, gs1)
    ssem = (ss0, ss1)

    _fill2d_f32(rows0, K, 0.0)

    def zbody(t, carry):
        pltpu.sync_copy(rows0, acc.at[pl.ds(s * ROWS_S + t * K, K), :])
        return carry

    lax.fori_loop(0, ROWS_S // K, zbody, 0)
    plsc.subcore_barrier()

    def start_idx(j, b):
        off = (j * NW + w) * K
        pltpu.async_copy(edge_hbm.at[0, pl.ds(off, K)], sidx[b], isem[b])
        pltpu.async_copy(edge_hbm.at[1, pl.ds(off, K)], didx[b], dsem[b])

    def wait_idx(b):
        pltpu.make_async_copy(edge_hbm.at[0, pl.ds(0, K)], sidx[b],
                              isem[b]).wait()
        pltpu.make_async_copy(edge_hbm.at[1, pl.ds(0, K)], didx[b],
                              dsem[b]).wait()

    def start_gather(b):
        pltpu.async_copy(y_hbm.at[sidx[b]], rows[b], gsem[b])

    def wait_gather(b):
        pltpu.make_async_copy(y_hbm.at[sidx[b]], rows[b], gsem[b]).wait()

    def sync_scatter(b):
        pltpu.sync_copy(rows[b], acc.at[didx[b]], add=True)

    nck = CH_W + jnp.where(w < TAIL, 1, 0)

    start_idx(0, 0)
    start_idx(1, 1)
    wait_idx(0)
    start_gather(0)

    def pair(p, carry):
        j0 = 2 * p
        wait_gather(0)
        wait_idx(1)
        start_gather(1)
        sync_scatter(0)

        @pl.when(j0 + 2 < nck)
        def _():
            start_idx(j0 + 2, 0)

        wait_gather(1)

        @pl.when(j0 + 2 < nck)
        def _():
            wait_idx(0)
            start_gather(0)

        sync_scatter(1)

        @pl.when(j0 + 3 < nck)
        def _():
            start_idx(j0 + 3, 1)

        return carry

    lax.fori_loop(0, CH_W // 2, pair, 0)

    @pl.when(w < TAIL)
    def _():
        wait_gather(0)
        sync_scatter(0)

    plsc.subcore_barrier()
    pltpu.sync_copy(acc.at[pl.ds(s * ROWS_S, ROWS_S), :],
                    out_hbm.at[c, pl.ds(s * ROWS_S, ROWS_S), :])


_agg_call = pl.kernel(
    _agg_body,
    out_type=jax.ShapeDtypeStruct((2, NP, D), jnp.float32),
    mesh=plsc.VectorSubcoreMesh(core_axis_name="c", subcore_axis_name="s"),
    scratch_types=[
        pltpu.VMEM_SHARED((NP, D), jnp.float32),
        pltpu.VMEM((K,), jnp.int32),
        pltpu.VMEM((K,), jnp.int32),
        pltpu.VMEM((K,), jnp.int32),
        pltpu.VMEM((K,), jnp.int32),
        pltpu.VMEM((K, D), jnp.float32),
        pltpu.VMEM((K, D), jnp.float32),
        pltpu.SemaphoreType.DMA,
        pltpu.SemaphoreType.DMA,
        pltpu.SemaphoreType.DMA,
        pltpu.SemaphoreType.DMA,
        pltpu.SemaphoreType.DMA,
        pltpu.SemaphoreType.DMA,
        pltpu.SemaphoreType.DMA,
        pltpu.SemaphoreType.DMA,
    ],
)


def _tc0_body(x_ref, w_ref, y_ref):
    y_ref[...] = jnp.dot(x_ref[...], w_ref[...], precision=_PREC,
                         preferred_element_type=jnp.float32)


def _tc0(xp, W1):
    return pl.pallas_call(
        _tc0_body,
        grid=(NB,),
        in_specs=[
            pl.BlockSpec((BN, D), lambda i: (i, 0)),
            pl.BlockSpec((D, H), lambda i: (0, 0)),
        ],
        out_specs=pl.BlockSpec((BN, H), lambda i: (i, 0)),
        out_shape=jax.ShapeDtypeStruct((NP, H), jnp.float32),
    )(xp, W1)


def _tc1b_body(degp, x_ref, w_ref, y_ref):
    deg = degp[0, :] + degp[1, :] + 1.0
    dinv = lax.rsqrt(deg)
    xw = jnp.dot(x_ref[...], w_ref[...], precision=_PREC,
                 preferred_element_type=jnp.float32)
    y_ref[...] = xw * dinv[:, None]


def _tc1b(degp, xp, W1):
    return pl.pallas_call(
        _tc1b_body,
        grid=(NB,),
        in_specs=[
            pl.BlockSpec((2, BN), lambda i: (0, i)),
            pl.BlockSpec((BN, D), lambda i: (i, 0)),
            pl.BlockSpec((D, H), lambda i: (0, 0)),
        ],
        out_specs=pl.BlockSpec((BN, H), lambda i: (i, 0)),
        out_shape=jax.ShapeDtypeStruct((NP, H), jnp.float32),
    )(degp, xp, W1)


def _tc2_body(degp, aggp, y_ref, b_ref, w_ref, out_ref):
    deg = degp[0, :] + degp[1, :] + 1.0
    dinv = lax.rsqrt(deg)
    agg = aggp[0] + aggp[1] + y_ref[...]
    h = jnp.maximum(agg * dinv[:, None] + b_ref[...], 0.0)
    hw = jnp.dot(h, w_ref[...], precision=_PREC,
                 preferred_element_type=jnp.float32)
    out_ref[...] = hw * dinv[:, None]


def _tc2(degp, aggp, y, b, W):
    return pl.pallas_call(
        _tc2_body,
        grid=(NB,),
        in_specs=[
            pl.BlockSpec((2, BN), lambda i: (0, i)),
            pl.BlockSpec((2, BN, D), lambda i: (0, i, 0)),
            pl.BlockSpec((BN, D), lambda i: (i, 0)),
            pl.BlockSpec((1, H), lambda i: (0, 0)),
            pl.BlockSpec((H, H), lambda i: (0, 0)),
        ],
        out_specs=pl.BlockSpec((BN, H), lambda i: (i, 0)),
        out_shape=jax.ShapeDtypeStruct((NP, H), jnp.float32),
    )(degp, aggp, y, b, W)


def _tc3_body(degp, aggp, y_ref, b_ref, wa_ref, ba_ref, wg_ref, bg_ref,
              batch_ref, wl_ref, bl_ref, out_ref,
              h2s, gates, m_acc, num_acc, den_acc):
    t = pl.program_id(0)
    i = pl.program_id(1)
    b = batch_ref[0, :]
    eq = b[:, None] == lax.broadcasted_iota(jnp.int32, (BN, G), 1)

    @pl.when(t == 0)
    def _():
        @pl.when(i == 0)
        def _():
            m_acc[...] = jnp.full((1, G), -jnp.inf, jnp.float32)

        deg = degp[0, :] + degp[1, :] + 1.0
        dinv = lax.rsqrt(deg)
        agg = aggp[0] + aggp[1] + y_ref[...]
        h2 = jnp.maximum(agg * dinv[:, None] + b_ref[...], 0.0)
        h2s[pl.ds(i * BN, BN), :] = h2
        a = jnp.maximum(jnp.dot(h2, wa_ref[...], precision=_PREC,
                                preferred_element_type=jnp.float32)
                        + ba_ref[...], 0.0)
        g = jnp.sum(a * wg_ref[...], axis=1) + bg_ref[0, 0]
        gates[0, pl.ds(i * BN, BN)] = g
        contrib = jnp.where(eq, g[:, None], -jnp.inf)
        m_acc[...] = jnp.maximum(m_acc[...], jnp.max(contrib, axis=0)[None, :])

    @pl.when(t == 1)
    def _():
        @pl.when(i == 0)
        def _():
            num_acc[...] = jnp.zeros((G, D), jnp.float32)
            den_acc[...] = jnp.zeros((G, D), jnp.float32)

        g = gates[0, pl.ds(i * BN, BN)]
        mm = m_acc[0, :]
        mm = jnp.where(mm == -jnp.inf, 0.0, mm)
        eqf = eq.astype(jnp.float32)
        mb = jnp.sum(eqf * mm[None, :], axis=1)
        e = jnp.where(b >= 0, jnp.exp(g - mb), 0.0)
        ew = eqf * e[:, None]
        dn = (((0,), (0,)), ((), ()))
        num_acc[...] += lax.dot_general(ew, h2s[pl.ds(i * BN, BN), :],
                                        dimension_numbers=dn, precision=_PREC,
                                        preferred_element_type=jnp.float32)
        den_acc[...] += lax.dot_general(ew, jnp.ones((BN, D), jnp.float32),
                                        dimension_numbers=dn, precision=_PREC,
                                        preferred_element_type=jnp.float32)

        @pl.when(i == NB - 1)
        def _():
            pooled = num_acc[...] / jnp.maximum(den_acc[...], 1e-16)
            out_ref[...] = jnp.dot(pooled, wl_ref[...], precision=_PREC,
                                   preferred_element_type=jnp.float32) \
                + bl_ref[...]


def _tc3(degp, aggp, y2, b2, Wa, ba, Wg, bg, batchp, Wl, bl):
    zero = lambda t, i: (0, 0)
    row = lambda t, i: (i, 0)
    rowt = lambda t, i: (0, i)
    return pl.pallas_call(
        _tc3_body,
        grid=(2, NB),
        in_specs=[
            pl.BlockSpec((2, BN), rowt),
            pl.BlockSpec((2, BN, D), lambda t, i: (0, i, 0)),
            pl.BlockSpec((BN, D), row),
            pl.BlockSpec((1, H), zero),
            pl.BlockSpec((H, H), zero),
            pl.BlockSpec((1, H), zero),
            pl.BlockSpec((1, H), zero),
            pl.BlockSpec((1, 1), zero),
            pl.BlockSpec((1, BN), rowt),
            pl.BlockSpec((H, C), zero),
            pl.BlockSpec((1, C), zero),
        ],
        out_specs=pl.BlockSpec((G, C), zero),
        out_shape=jax.ShapeDtypeStruct((G, C), jnp.float32),
        scratch_shapes=[pltpu.VMEM((NP, D), jnp.float32),
                        pltpu.VMEM((1, NP), jnp.float32),
                        pltpu.VMEM((1, G), jnp.float32),
                        pltpu.VMEM((G, D), jnp.float32),
                        pltpu.VMEM((G, D), jnp.float32)],
    )(degp, aggp, y2, b2, Wa, ba, Wg, bg, batchp, Wl, bl)


@jax.jit
def kernel(x, edge_index, batch, W1, b1, W2, b2, Wa, ba, Wg, bg, Wl, bl):
    xp = jnp.pad(x, ((0, NP - N), (0, 0)))
    batchp = jnp.pad(batch, (0, NP - N), constant_values=-1).reshape(1, NP)

    degp = _deg_call(edge_index)
    y1 = _tc1b(degp, xp, W1)
    aggp1 = _agg_call(y1, edge_index)
    y2 = _tc2(degp, aggp1, y1, b1.reshape(1, H), W2)
    aggp2 = _agg_call(y2, edge_index)
    out = _tc3(degp, aggp2, y2, b2.reshape(1, H), Wa, ba.reshape(1, H),
               Wg.reshape(1, H), bg.reshape(1, 1), batchp, Wl,
               bl.reshape(1, C))
    return out


# speedup vs baseline: 1.0050x; 1.0050x over previous
import functools

import jax
import jax.numpy as jnp
from jax import lax
from jax.experimental import pallas as pl
from jax.experimental.pallas import tpu as pltpu
from jax.experimental.pallas import tpu_sc as plsc

N = 10000
E = 320000
D = 128
H = 128
C = 10
G = 128

NP = 10240
NW = 32
ROWS_S = NP // 16
K = 128
NCHUNK = E // K
CH_W = NCHUNK // NW
TAIL = NCHUNK - CH_W * NW

BN = 1024
NB = NP // BN

_PREC = lax.Precision.HIGHEST


def _fill_f32(ref, n, val):
    def body(i, c):
        ref[pl.ds(i * 16, 16)] = jnp.full((16,), val, jnp.float32)
        return c
    lax.fori_loop(0, n // 16, body, 0)


def _fill2d_f32(ref, rows, val):
    def body(k, c):
        ref[k // 8, pl.ds((k % 8) * 16, 16)] = jnp.full((16,), val, jnp.float32)
        return c
    lax.fori_loop(0, rows * 8, body, 0)


def _deg_body(edge_hbm, out_hbm, acc, ones_v, zv, idx0, idx1, i0, i1):
    c = lax.axis_index("c")
    s = lax.axis_index("s")
    w = c * 16 + s
    idx = (idx0, idx1)
    isem = (i0, i1)
    _fill_f32(ones_v, K, 1.0)
    _fill_f32(zv, ROWS_S, 0.0)
    nck = CH_W + jnp.where(w < TAIL, 1, 0)

    def start_idx(j, b):
        off = (j * NW + w) * K
        pltpu.async_copy(edge_hbm.at[1, pl.ds(off, K)], idx[b], isem[b])

    def wait_idx(b):
        pltpu.make_async_copy(edge_hbm.at[1, pl.ds(0, K)], idx[b],
                              isem[b]).wait()

    start_idx(0, 0)
    start_idx(1, 1)
    pltpu.sync_copy(zv, acc.at[pl.ds(s * ROWS_S, ROWS_S)])
    plsc.subcore_barrier()

    def pair(p, carry):
        j0 = 2 * p
        wait_idx(0)
        pltpu.sync_copy(ones_v, acc.at[idx[0]], add=True)

        @pl.when(j0 + 2 < nck)
        def _():
            start_idx(j0 + 2, 0)

        wait_idx(1)
        pltpu.sync_copy(ones_v, acc.at[idx[1]], add=True)

        @pl.when(j0 + 3 < nck)
        def _():
            start_idx(j0 + 3, 1)

        return carry

    lax.fori_loop(0, CH_W // 2, pair, 0)

    @pl.when(w < TAIL)
    def _():
        wait_idx(0)
        pltpu.sync_copy(ones_v, acc.at[idx[0]], add=True)

    plsc.subcore_barrier()
    pltpu.sync_copy(acc.at[pl.ds(s * ROWS_S, ROWS_S)],
                    out_hbm.at[c, pl.ds(s * ROWS_S, ROWS_S)])


_deg_call = pl.kernel(
    _deg_body,
    out_type=jax.ShapeDtypeStruct((2, NP), jnp.float32),
    mesh=plsc.VectorSubcoreMesh(core_axis_name="c", subcore_axis_name="s"),
    scratch_types=[
        pltpu.VMEM_SHARED((NP,), jnp.float32),
        pltpu.VMEM((K,), jnp.float32),
        pltpu.VMEM((ROWS_S,), jnp.float32),
        pltpu.VMEM((K,), jnp.int32),
        pltpu.VMEM((K,), jnp.int32),
        pltpu.SemaphoreType.DMA,
        pltpu.SemaphoreType.DMA,
    ],
)


def _agg_body(y_hbm, edge_hbm, out_hbm, acc,
              sidx0, sidx1, didx0, didx1, rows0, rows1,
              is0, is1, id0, id1, gs0, gs1, ss0, ss1):
    c = lax.axis_index("c")
    s = lax.axis_index("s")
    w = c * 16 + s
    sidx = (sidx0, sidx1)
    didx = (didx0, didx1)
    rows = (rows0, rows1)
    isem = (is0, is1)
    dsem = (id0, id1)
    gsem = (gs0, gs1)
    ssem = (ss0, ss1)

    _fill2d_f32(rows0, K, 0.0)
    for t in range(ROWS_S // K):
        pltpu.async_copy(rows0, acc.at[pl.ds(s * ROWS_S + t * K, K), :], gs0)

    def start_idx(j, b):
        off = (j * NW + w) * K
        pltpu.async_copy(edge_hbm.at[0, pl.ds(off, K)], sidx[b], isem[b])
        pltpu.async_copy(edge_hbm.at[1, pl.ds(off, K)], didx[b], dsem[b])

    def wait_idx(b):
        pltpu.make_async_copy(edge_hbm.at[0, pl.ds(0, K)], sidx[b],
                              isem[b]).wait()
        pltpu.make_async_copy(edge_hbm.at[1, pl.ds(0, K)], didx[b],
                              dsem[b]).wait()

    def start_gather(b):
        pltpu.async_copy(y_hbm.at[sidx[b]], rows[b], gsem[b])

    def wait_gather(b):
        pltpu.make_async_copy(y_hbm.at[sidx[b]], rows[b], gsem[b]).wait()

    def sync_scatter(b):
        pltpu.sync_copy(rows[b], acc.at[didx[b]], add=True)

    nck = CH_W + jnp.where(w < TAIL, 1, 0)

    start_idx(0, 0)
    start_idx(1, 1)
    for t in range(ROWS_S // K):
        pltpu.make_async_copy(rows0,
                              acc.at[pl.ds(s * ROWS_S + t * K, K), :],
                              gs0).wait()
    plsc.subcore_barrier()
    wait_idx(0)
    start_gather(0)

    def pair(p, carry):
        j0 = 2 * p
        wait_gather(0)
        wait_idx(1)
        start_gather(1)
        sync_scatter(0)

        @pl.when(j0 + 2 < nck)
        def _():
            start_idx(j0 + 2, 0)

        wait_gather(1)

        @pl.when(j0 + 2 < nck)
        def _():
            wait_idx(0)
            start_gather(0)

        sync_scatter(1)

        @pl.when(j0 + 3 < nck)
        def _():
            start_idx(j0 + 3, 1)

        return carry

    lax.fori_loop(0, CH_W // 2, pair, 0)

    @pl.when(w < TAIL)
    def _():
        wait_gather(0)
        sync_scatter(0)

    plsc.subcore_barrier()
    pltpu.sync_copy(acc.at[pl.ds(s * ROWS_S, ROWS_S), :],
                    out_hbm.at[c, pl.ds(s * ROWS_S, ROWS_S), :])


_agg_call = pl.kernel(
    _agg_body,
    out_type=jax.ShapeDtypeStruct((2, NP, D), jnp.float32),
    mesh=plsc.VectorSubcoreMesh(core_axis_name="c", subcore_axis_name="s"),
    scratch_types=[
        pltpu.VMEM_SHARED((NP, D), jnp.float32),
        pltpu.VMEM((K,), jnp.int32),
        pltpu.VMEM((K,), jnp.int32),
        pltpu.VMEM((K,), jnp.int32),
        pltpu.VMEM((K,), jnp.int32),
        pltpu.VMEM((K, D), jnp.float32),
        pltpu.VMEM((K, D), jnp.float32),
        pltpu.SemaphoreType.DMA,
        pltpu.SemaphoreType.DMA,
        pltpu.SemaphoreType.DMA,
        pltpu.SemaphoreType.DMA,
        pltpu.SemaphoreType.DMA,
        pltpu.SemaphoreType.DMA,
        pltpu.SemaphoreType.DMA,
        pltpu.SemaphoreType.DMA,
    ],
)


def _tc1b_body(degp, x_ref, w_ref, y_ref):
    deg = degp[0, :] + degp[1, :] + 1.0
    dinv = lax.rsqrt(deg)
    xw = jnp.dot(x_ref[...], w_ref[...], precision=_PREC,
                 preferred_element_type=jnp.float32)
    y_ref[...] = xw * dinv[:, None]


def _tc1b(degp, xp, W1):
    return pl.pallas_call(
        _tc1b_body,
        grid=(NB,),
        in_specs=[
            pl.BlockSpec((2, BN), lambda i: (0, i)),
            pl.BlockSpec((BN, D), lambda i: (i, 0)),
            pl.BlockSpec((D, H), lambda i: (0, 0)),
        ],
        out_specs=pl.BlockSpec((BN, H), lambda i: (i, 0)),
        out_shape=jax.ShapeDtypeStruct((NP, H), jnp.float32),
    )(degp, xp, W1)


def _tc2_body(degp, aggp, y_ref, b_ref, w_ref, out_ref):
    deg = degp[0, :] + degp[1, :] + 1.0
    dinv = lax.rsqrt(deg)
    agg = aggp[0] + aggp[1] + y_ref[...]
    h = jnp.maximum(agg * dinv[:, None] + b_ref[...], 0.0)
    hw = jnp.dot(h, w_ref[...], precision=_PREC,
                 preferred_element_type=jnp.float32)
    out_ref[...] = hw * dinv[:, None]


def _tc2(degp, aggp, y, b, W):
    return pl.pallas_call(
        _tc2_body,
        grid=(NB,),
        in_specs=[
            pl.BlockSpec((2, BN), lambda i: (0, i)),
            pl.BlockSpec((2, BN, D), lambda i: (0, i, 0)),
            pl.BlockSpec((BN, D), lambda i: (i, 0)),
            pl.BlockSpec((1, H), lambda i: (0, 0)),
            pl.BlockSpec((H, H), lambda i: (0, 0)),
        ],
        out_specs=pl.BlockSpec((BN, H), lambda i: (i, 0)),
        out_shape=jax.ShapeDtypeStruct((NP, H), jnp.float32),
    )(degp, aggp, y, b, W)


def _tc3_body(degp, aggp, y_ref, b_ref, wa_ref, ba_ref, wg_ref, bg_ref,
              batch_ref, wl_ref, bl_ref, out_ref,
              h2s, gates, m_acc, num_acc, den_acc):
    t = pl.program_id(0)
    i = pl.program_id(1)
    b = batch_ref[0, :]
    eq = b[:, None] == lax.broadcasted_iota(jnp.int32, (BN, G), 1)

    @pl.when(t == 0)
    def _():
        @pl.when(i == 0)
        def _():
            m_acc[...] = jnp.full((1, G), -jnp.inf, jnp.float32)

        deg = degp[0, :] + degp[1, :] + 1.0
        dinv = lax.rsqrt(deg)
        agg = aggp[0] + aggp[1] + y_ref[...]
        h2 = jnp.maximum(agg * dinv[:, None] + b_ref[...], 0.0)
        h2s[pl.ds(i * BN, BN), :] = h2
        a = jnp.maximum(jnp.dot(h2, wa_ref[...], precision=_PREC,
                                preferred_element_type=jnp.float32)
                        + ba_ref[...], 0.0)
        g = jnp.sum(a * wg_ref[...], axis=1) + bg_ref[0, 0]
        gates[0, pl.ds(i * BN, BN)] = g
        contrib = jnp.where(eq, g[:, None], -jnp.inf)
        m_acc[...] = jnp.maximum(m_acc[...], jnp.max(contrib, axis=0)[None, :])

    @pl.when(t == 1)
    def _():
        @pl.when(i == 0)
        def _():
            num_acc[...] = jnp.zeros((G, D), jnp.float32)
            den_acc[...] = jnp.zeros((G, D), jnp.float32)

        g = gates[0, pl.ds(i * BN, BN)]
        mm = m_acc[0, :]
        mm = jnp.where(mm == -jnp.inf, 0.0, mm)
        eqf = eq.astype(jnp.float32)
        mb = jnp.sum(eqf * mm[None, :], axis=1)
        e = jnp.where(b >= 0, jnp.exp(g - mb), 0.0)
        ew = eqf * e[:, None]
        dn = (((0,), (0,)), ((), ()))
        num_acc[...] += lax.dot_general(ew, h2s[pl.ds(i * BN, BN), :],
                                        dimension_numbers=dn, precision=_PREC,
                                        preferred_element_type=jnp.float32)
        den_acc[...] += lax.dot_general(ew, jnp.ones((BN, D), jnp.float32),
                                        dimension_numbers=dn, precision=_PREC,
                                        preferred_element_type=jnp.float32)

        @pl.when(i == NB - 1)
        def _():
            pooled = num_acc[...] / jnp.maximum(den_acc[...], 1e-16)
            out_ref[...] = jnp.dot(pooled, wl_ref[...], precision=_PREC,
                                   preferred_element_type=jnp.float32) \
                + bl_ref[...]


def _tc3(degp, aggp, y2, b2, Wa, ba, Wg, bg, batchp, Wl, bl):
    zero = lambda t, i: (0, 0)
    row = lambda t, i: (i, 0)
    rowt = lambda t, i: (0, i)
    return pl.pallas_call(
        _tc3_body,
        grid=(2, NB),
        in_specs=[
            pl.BlockSpec((2, BN), rowt),
            pl.BlockSpec((2, BN, D), lambda t, i: (0, i, 0)),
            pl.BlockSpec((BN, D), row),
            pl.BlockSpec((1, H), zero),
            pl.BlockSpec((H, H), zero),
            pl.BlockSpec((1, H), zero),
            pl.BlockSpec((1, H), zero),
            pl.BlockSpec((1, 1), zero),
            pl.BlockSpec((1, BN), rowt),
            pl.BlockSpec((H, C), zero),
            pl.BlockSpec((1, C), zero),
        ],
        out_specs=pl.BlockSpec((G, C), zero),
        out_shape=jax.ShapeDtypeStruct((G, C), jnp.float32),
        scratch_shapes=[pltpu.VMEM((NP, D), jnp.float32),
                        pltpu.VMEM((1, NP), jnp.float32),
                        pltpu.VMEM((1, G), jnp.float32),
                        pltpu.VMEM((G, D), jnp.float32),
                        pltpu.VMEM((G, D), jnp.float32)],
    )(degp, aggp, y2, b2, Wa, ba, Wg, bg, batchp, Wl, bl)


@jax.jit
def kernel(x, edge_index, batch, W1, b1, W2, b2, Wa, ba, Wg, bg, Wl, bl):
    xp = jnp.pad(x, ((0, NP - N), (0, 0)))
    batchp = jnp.pad(batch, (0, NP - N), constant_values=-1).reshape(1, NP)

    degp = _deg_call(edge_index)
    y1 = _tc1b(degp, xp, W1)
    aggp1 = _agg_call(y1, edge_index)
    y2 = _tc2(degp, aggp1, y1, b1.reshape(1, H), W2)
    aggp2 = _agg_call(y2, edge_index)
    out = _tc3(degp, aggp2, y2, b2.reshape(1, H), Wa, ba.reshape(1, H),
               Wg.reshape(1, H), bg.reshape(1, 1), batchp, Wl,
               bl.reshape(1, C))
    return out


# speedup vs baseline: 1.0072x; 1.0022x over previous
import functools

import jax
import jax.numpy as jnp
from jax import lax
from jax.experimental import pallas as pl
from jax.experimental.pallas import tpu as pltpu
from jax.experimental.pallas import tpu_sc as plsc

N = 10000
E = 320000
D = 128
H = 128
C = 10
G = 128

NP = 10240
NW = 32
ROWS_S = NP // 16
K = 128
NCHUNK = E // K
CH_W = NCHUNK // NW
TAIL = NCHUNK - CH_W * NW

BN = 1024
NB = NP // BN

_PREC = lax.Precision.HIGHEST


def _fill_f32(ref, n, val):
    def body(i, c):
        ref[pl.ds(i * 16, 16)] = jnp.full((16,), val, jnp.float32)
        return c
    lax.fori_loop(0, n // 16, body, 0)


def _fill2d_f32(ref, rows, val):
    def body(k, c):
        ref[k // 8, pl.ds((k % 8) * 16, 16)] = jnp.full((16,), val, jnp.float32)
        return c
    lax.fori_loop(0, rows * 8, body, 0)


def _deg_body(edge_hbm, out_hbm, acc, ones_v, zv, idx0, idx1, i0, i1):
    c = lax.axis_index("c")
    s = lax.axis_index("s")
    w = c * 16 + s
    idx = (idx0, idx1)
    isem = (i0, i1)
    _fill_f32(ones_v, K, 1.0)
    _fill_f32(zv, ROWS_S, 0.0)
    nck = CH_W + jnp.where(w < TAIL, 1, 0)

    def start_idx(j, b):
        off = (j * NW + w) * K
        pltpu.async_copy(edge_hbm.at[1, pl.ds(off, K)], idx[b], isem[b])

    def wait_idx(b):
        pltpu.make_async_copy(edge_hbm.at[1, pl.ds(0, K)], idx[b],
                              isem[b]).wait()

    start_idx(0, 0)
    start_idx(1, 1)
    pltpu.sync_copy(zv, acc.at[pl.ds(s * ROWS_S, ROWS_S)])
    plsc.subcore_barrier()

    def pair(p, carry):
        j0 = 2 * p
        wait_idx(0)
        pltpu.sync_copy(ones_v, acc.at[idx[0]], add=True)

        @pl.when(j0 + 2 < nck)
        def _():
            start_idx(j0 + 2, 0)

        wait_idx(1)
        pltpu.sync_copy(ones_v, acc.at[idx[1]], add=True)

        @pl.when(j0 + 3 < nck)
        def _():
            start_idx(j0 + 3, 1)

        return carry

    lax.fori_loop(0, CH_W // 2, pair, 0)

    @pl.when(w < TAIL)
    def _():
        wait_idx(0)
        pltpu.sync_copy(ones_v, acc.at[idx[0]], add=True)

    plsc.subcore_barrier()
    pltpu.sync_copy(acc.at[pl.ds(s * ROWS_S, ROWS_S)],
                    out_hbm.at[c, pl.ds(s * ROWS_S, ROWS_S)])


_deg_call = pl.kernel(
    _deg_body,
    out_type=jax.ShapeDtypeStruct((2, NP), jnp.float32),
    mesh=plsc.VectorSubcoreMesh(core_axis_name="c", subcore_axis_name="s"),
    scratch_types=[
        pltpu.VMEM_SHARED((NP,), jnp.float32),
        pltpu.VMEM((K,), jnp.float32),
        pltpu.VMEM((ROWS_S,), jnp.float32),
        pltpu.VMEM((K,), jnp.int32),
        pltpu.VMEM((K,), jnp.int32),
        pltpu.SemaphoreType.DMA,
        pltpu.SemaphoreType.DMA,
    ],
)


def _agg_body(y_hbm, edge_hbm, out_hbm, acc,
              sidx0, sidx1, didx0, didx1, sdidx0, sdidx1, rows0, rows1,
              is0, is1, id0, id1, gs0, gs1, ss0, ss1):
    c = lax.axis_index("c")
    s = lax.axis_index("s")
    w = c * 16 + s
    sidx = (sidx0, sidx1)
    didx = (didx0, didx1)
    sdidx = (sdidx0, sdidx1)
    rows = (rows0, rows1)
    isem = (is0, is1)
    dsem = (id0, id1)
    gsem = (gs0, gs1)
    ssem = (ss0, ss1)

    _fill2d_f32(rows0, K, 0.0)
    for t in range(ROWS_S // K):
        pltpu.async_copy(rows0, acc.at[pl.ds(s * ROWS_S + t * K, K), :], gs0)

    def start_idx(j, b):
        off = (j * NW + w) * K
        pltpu.async_copy(edge_hbm.at[0, pl.ds(off, K)], sidx[b], isem[b])
        pltpu.async_copy(edge_hbm.at[1, pl.ds(off, K)], didx[b], dsem[b])

    def wait_idx(b):
        pltpu.make_async_copy(edge_hbm.at[0, pl.ds(0, K)], sidx[b],
                              isem[b]).wait()
        pltpu.make_async_copy(edge_hbm.at[1, pl.ds(0, K)], didx[b],
                              dsem[b]).wait()

    def start_gather(b):
        pltpu.async_copy(y_hbm.at[sidx[b]], rows[b], gsem[b])

    def wait_gather(b):
        pltpu.make_async_copy(y_hbm.at[sidx[b]], rows[b], gsem[b]).wait()

    def copy_didx(b):
        for t in range(K // 16):
            sdidx[b][pl.ds(t * 16, 16)] = didx[b][pl.ds(t * 16, 16)]

    def start_scatter(b):
        pltpu.async_copy(rows[b], acc.at[sdidx[b]], ssem[b], add=True)

    def wait_scatter(b):
        pltpu.make_async_copy(rows[b], acc.at[sdidx[b]], ssem[b]).wait()

    nck = CH_W + jnp.where(w < TAIL, 1, 0)

    start_idx(0, 0)
    start_idx(1, 1)
    for t in range(ROWS_S // K):
        pltpu.make_async_copy(rows0,
                              acc.at[pl.ds(s * ROWS_S + t * K, K), :],
                              gs0).wait()
    plsc.subcore_barrier()
    wait_idx(0)
    start_gather(0)

    def pair(p, carry):
        j0 = 2 * p
        wait_gather(0)

        @pl.when(p > 0)
        def _():
            wait_scatter(1)

        wait_idx(1)
        start_gather(1)
        copy_didx(0)
        start_scatter(0)

        @pl.when(j0 + 2 < nck)
        def _():
            start_idx(j0 + 2, 0)

        wait_gather(1)
        wait_scatter(0)

        @pl.when(j0 + 2 < nck)
        def _():
            wait_idx(0)
            start_gather(0)

        copy_didx(1)
        start_scatter(1)

        @pl.when(j0 + 3 < nck)
        def _():
            start_idx(j0 + 3, 1)

        return carry

    lax.fori_loop(0, CH_W // 2, pair, 0)

    wait_scatter(1)

    @pl.when(w < TAIL)
    def _():
        wait_gather(0)
        copy_didx(0)
        start_scatter(0)
        wait_scatter(0)

    plsc.subcore_barrier()
    pltpu.sync_copy(acc.at[pl.ds(s * ROWS_S, ROWS_S), :],
                    out_hbm.at[c, pl.ds(s * ROWS_S, ROWS_S), :])


_agg_call = pl.kernel(
    _agg_body,
    out_type=jax.ShapeDtypeStruct((2, NP, D), jnp.float32),
    mesh=plsc.VectorSubcoreMesh(core_axis_name="c", subcore_axis_name="s"),
    scratch_types=[
        pltpu.VMEM_SHARED((NP, D), jnp.float32),
        pltpu.VMEM((K,), jnp.int32),
        pltpu.VMEM((K,), jnp.int32),
        pltpu.VMEM((K,), jnp.int32),
        pltpu.VMEM((K,), jnp.int32),
        pltpu.VMEM((K,), jnp.int32),
        pltpu.VMEM((K,), jnp.int32),
        pltpu.VMEM((K, D), jnp.float32),
        pltpu.VMEM((K, D), jnp.float32),
        pltpu.SemaphoreType.DMA,
        pltpu.SemaphoreType.DMA,
        pltpu.SemaphoreType.DMA,
        pltpu.SemaphoreType.DMA,
        pltpu.SemaphoreType.DMA,
        pltpu.SemaphoreType.DMA,
        pltpu.SemaphoreType.DMA,
        pltpu.SemaphoreType.DMA,
    ],
)


def _tc1b_body(degp, x_ref, w_ref, y_ref):
    deg = degp[0, :] + degp[1, :] + 1.0
    dinv = lax.rsqrt(deg)
    xw = jnp.dot(x_ref[...], w_ref[...], precision=_PREC,
                 preferred_element_type=jnp.float32)
    y_ref[...] = xw * dinv[:, None]


def _tc1b(degp, xp, W1):
    return pl.pallas_call(
        _tc1b_body,
        grid=(NB,),
        in_specs=[
            pl.BlockSpec((2, BN), lambda i: (0, i)),
            pl.BlockSpec((BN, D), lambda i: (i, 0)),
            pl.BlockSpec((D, H), lambda i: (0, 0)),
        ],
        out_specs=pl.BlockSpec((BN, H), lambda i: (i, 0)),
        out_shape=jax.ShapeDtypeStruct((NP, H), jnp.float32),
    )(degp, xp, W1)


def _tc2_body(degp, aggp, y_ref, b_ref, w_ref, out_ref):
    deg = degp[0, :] + degp[1, :] + 1.0
    dinv = lax.rsqrt(deg)
    agg = aggp[0] + aggp[1] + y_ref[...]
    h = jnp.maximum(agg * dinv[:, None] + b_ref[...], 0.0)
    hw = jnp.dot(h, w_ref[...], precision=_PREC,
                 preferred_element_type=jnp.float32)
    out_ref[...] = hw * dinv[:, None]


def _tc2(degp, aggp, y, b, W):
    return pl.pallas_call(
        _tc2_body,
        grid=(NB,),
        in_specs=[
            pl.BlockSpec((2, BN), lambda i: (0, i)),
            pl.BlockSpec((2, BN, D), lambda i: (0, i, 0)),
            pl.BlockSpec((BN, D), lambda i: (i, 0)),
            pl.BlockSpec((1, H), lambda i: (0, 0)),
            pl.BlockSpec((H, H), lambda i: (0, 0)),
        ],
        out_specs=pl.BlockSpec((BN, H), lambda i: (i, 0)),
        out_shape=jax.ShapeDtypeStruct((NP, H), jnp.float32),
    )(degp, aggp, y, b, W)


def _tc3_body(degp, aggp, y_ref, b_ref, wa_ref, ba_ref, wg_ref, bg_ref,
              batch_ref, wl_ref, bl_ref, out_ref,
              h2s, gates, m_acc, num_acc, den_acc):
    t = pl.program_id(0)
    i = pl.program_id(1)
    b = batch_ref[0, :]
    eq = b[:, None] == lax.broadcasted_iota(jnp.int32, (BN, G), 1)

    @pl.when(t == 0)
    def _():
        @pl.when(i == 0)
        def _():
            m_acc[...] = jnp.full((1, G), -jnp.inf, jnp.float32)

        deg = degp[0, :] + degp[1, :] + 1.0
        dinv = lax.rsqrt(deg)
        agg = aggp[0] + aggp[1] + y_ref[...]
        h2 = jnp.maximum(agg * dinv[:, None] + b_ref[...], 0.0)
        h2s[pl.ds(i * BN, BN), :] = h2
        a = jnp.maximum(jnp.dot(h2, wa_ref[...], precision=_PREC,
                                preferred_element_type=jnp.float32)
                        + ba_ref[...], 0.0)
        g = jnp.sum(a * wg_ref[...], axis=1) + bg_ref[0, 0]
        gates[0, pl.ds(i * BN, BN)] = g
        contrib = jnp.where(eq, g[:, None], -jnp.inf)
        m_acc[...] = jnp.maximum(m_acc[...], jnp.max(contrib, axis=0)[None, :])

    @pl.when(t == 1)
    def _():
        @pl.when(i == 0)
        def _():
            num_acc[...] = jnp.zeros((G, D), jnp.float32)
            den_acc[...] = jnp.zeros((G, D), jnp.float32)

        g = gates[0, pl.ds(i * BN, BN)]
        mm = m_acc[0, :]
        mm = jnp.where(mm == -jnp.inf, 0.0, mm)
        eqf = eq.astype(jnp.float32)
        mb = jnp.sum(eqf * mm[None, :], axis=1)
        e = jnp.where(b >= 0, jnp.exp(g - mb), 0.0)
        ew = eqf * e[:, None]
        dn = (((0,), (0,)), ((), ()))
        num_acc[...] += lax.dot_general(ew, h2s[pl.ds(i * BN, BN), :],
                                        dimension_numbers=dn, precision=_PREC,
                                        preferred_element_type=jnp.float32)
        den_acc[...] += lax.dot_general(ew, jnp.ones((BN, D), jnp.float32),
                                        dimension_numbers=dn, precision=_PREC,
                                        preferred_element_type=jnp.float32)

        @pl.when(i == NB - 1)
        def _():
            pooled = num_acc[...] / jnp.maximum(den_acc[...], 1e-16)
            out_ref[...] = jnp.dot(pooled, wl_ref[...], precision=_PREC,
                                   preferred_element_type=jnp.float32) \
                + bl_ref[...]


def _tc3(degp, aggp, y2, b2, Wa, ba, Wg, bg, batchp, Wl, bl):
    zero = lambda t, i: (0, 0)
    row = lambda t, i: (i, 0)
    rowt = lambda t, i: (0, i)
    return pl.pallas_call(
        _tc3_body,
        grid=(2, NB),
        in_specs=[
            pl.BlockSpec((2, BN), rowt),
            pl.BlockSpec((2, BN, D), lambda t, i: (0, i, 0)),
            pl.BlockSpec((BN, D), row),
            pl.BlockSpec((1, H), zero),
            pl.BlockSpec((H, H), zero),
            pl.BlockSpec((1, H), zero),
            pl.BlockSpec((1, H), zero),
            pl.BlockSpec((1, 1), zero),
            pl.BlockSpec((1, BN), rowt),
            pl.BlockSpec((H, C), zero),
            pl.BlockSpec((1, C), zero),
        ],
        out_specs=pl.BlockSpec((G, C), zero),
        out_shape=jax.ShapeDtypeStruct((G, C), jnp.float32),
        scratch_shapes=[pltpu.VMEM((NP, D), jnp.float32),
                        pltpu.VMEM((1, NP), jnp.float32),
                        pltpu.VMEM((1, G), jnp.float32),
                        pltpu.VMEM((G, D), jnp.float32),
                        pltpu.VMEM((G, D), jnp.float32)],
    )(degp, aggp, y2, b2, Wa, ba, Wg, bg, batchp, Wl, bl)


@jax.jit
def kernel(x, edge_index, batch, W1, b1, W2, b2, Wa, ba, Wg, bg, Wl, bl):
    xp = jnp.pad(x, ((0, NP - N), (0, 0)))
    batchp = jnp.pad(batch, (0, NP - N), constant_values=-1).reshape(1, NP)

    degp = _deg_call(edge_index)
    y1 = _tc1b(degp, xp, W1)
    aggp1 = _agg_call(y1, edge_index)
    y2 = _tc2(degp, aggp1, y1, b1.reshape(1, H), W2)
    aggp2 = _agg_call(y2, edge_index)
    out = _tc3(degp, aggp2, y2, b2.reshape(1, H), Wa, ba.reshape(1, H),
               Wg.reshape(1, H), bg.reshape(1, 1), batchp, Wl,
               bl.reshape(1, C))
    return out


# speedup vs baseline: 1.0112x; 1.0040x over previous
import functools

import jax
import jax.numpy as jnp
from jax import lax
from jax.experimental import pallas as pl
from jax.experimental.pallas import tpu as pltpu
from jax.experimental.pallas import tpu_sc as plsc

N = 10000
E = 320000
D = 128
H = 128
C = 10
G = 128

NP = 10240
NW = 32
ROWS_S = NP // 16
K = 128
NCHUNK = E // K
CH_W = NCHUNK // NW
TAIL = NCHUNK - CH_W * NW

BN = 1024
NB = NP // BN

_PREC = lax.Precision.HIGHEST


def _fill_f32(ref, n, val):
    def body(i, c):
        ref[pl.ds(i * 16, 16)] = jnp.full((16,), val, jnp.float32)
        return c
    lax.fori_loop(0, n // 16, body, 0)


def _fill2d_f32(ref, rows, val):
    def body(k, c):
        ref[k // 8, pl.ds((k % 8) * 16, 16)] = jnp.full((16,), val, jnp.float32)
        return c
    lax.fori_loop(0, rows * 8, body, 0)


def _deg_body(edge_hbm, out_hbm, acc, ones_v, zv, idx0, idx1, sx0, sx1,
              i0, i1, s0, s1):
    c = lax.axis_index("c")
    s = lax.axis_index("s")
    w = c * 16 + s
    idx = (idx0, idx1)
    sidx = (sx0, sx1)
    isem = (i0, i1)
    ssem = (s0, s1)
    _fill_f32(ones_v, K, 1.0)
    _fill_f32(zv, ROWS_S, 0.0)
    nck = CH_W + jnp.where(w < TAIL, 1, 0)

    def start_idx(j, b):
        off = (j * NW + w) * K
        pltpu.async_copy(edge_hbm.at[1, pl.ds(off, K)], idx[b], isem[b])

    def wait_idx(b):
        pltpu.make_async_copy(edge_hbm.at[1, pl.ds(0, K)], idx[b],
                              isem[b]).wait()

    def copy_idx(b):
        for t in range(K // 16):
            sidx[b][pl.ds(t * 16, 16)] = idx[b][pl.ds(t * 16, 16)]

    def start_scatter(b):
        pltpu.async_copy(ones_v, acc.at[sidx[b]], ssem[b], add=True)

    def wait_scatter(b):
        pltpu.make_async_copy(ones_v, acc.at[sidx[b]], ssem[b]).wait()

    start_idx(0, 0)
    start_idx(1, 1)
    pltpu.sync_copy(zv, acc.at[pl.ds(s * ROWS_S, ROWS_S)])
    plsc.subcore_barrier()

    def pair(p, carry):
        j0 = 2 * p
        wait_idx(0)

        @pl.when(p > 0)
        def _():
            wait_scatter(0)

        copy_idx(0)
        start_scatter(0)

        @pl.when(j0 + 2 < nck)
        def _():
            start_idx(j0 + 2, 0)

        wait_idx(1)

        @pl.when(p > 0)
        def _():
            wait_scatter(1)

        copy_idx(1)
        start_scatter(1)

        @pl.when(j0 + 3 < nck)
        def _():
            start_idx(j0 + 3, 1)

        return carry

    lax.fori_loop(0, CH_W // 2, pair, 0)
    wait_scatter(0)
    wait_scatter(1)

    @pl.when(w < TAIL)
    def _():
        wait_idx(0)
        copy_idx(0)
        start_scatter(0)
        wait_scatter(0)

    plsc.subcore_barrier()
    pltpu.sync_copy(acc.at[pl.ds(s * ROWS_S, ROWS_S)],
                    out_hbm.at[c, pl.ds(s * ROWS_S, ROWS_S)])


_deg_call = pl.kernel(
    _deg_body,
    out_type=jax.ShapeDtypeStruct((2, NP), jnp.float32),
    mesh=plsc.VectorSubcoreMesh(core_axis_name="c", subcore_axis_name="s"),
    scratch_types=[
        pltpu.VMEM_SHARED((NP,), jnp.float32),
        pltpu.VMEM((K,), jnp.float32),
        pltpu.VMEM((ROWS_S,), jnp.float32),
        pltpu.VMEM((K,), jnp.int32),
        pltpu.VMEM((K,), jnp.int32),
        pltpu.VMEM((K,), jnp.int32),
        pltpu.VMEM((K,), jnp.int32),
        pltpu.SemaphoreType.DMA,
        pltpu.SemaphoreType.DMA,
        pltpu.SemaphoreType.DMA,
        pltpu.SemaphoreType.DMA,
    ],
)


def _agg_body(y_hbm, edge_hbm, out_hbm, acc,
              sidx0, sidx1, didx0, didx1, sdidx0, sdidx1, rows0, rows1,
              is0, is1, id0, id1, gs0, gs1, ss0, ss1):
    c = lax.axis_index("c")
    s = lax.axis_index("s")
    w = c * 16 + s
    sidx = (sidx0, sidx1)
    didx = (didx0, didx1)
    sdidx = (sdidx0, sdidx1)
    rows = (rows0, rows1)
    isem = (is0, is1)
    dsem = (id0, id1)
    gsem = (gs0, gs1)
    ssem = (ss0, ss1)

    _fill2d_f32(rows0, K, 0.0)
    for t in range(ROWS_S // K):
        pltpu.async_copy(rows0, acc.at[pl.ds(s * ROWS_S + t * K, K), :], gs0)

    def start_idx(j, b):
        off = (j * NW + w) * K
        pltpu.async_copy(edge_hbm.at[0, pl.ds(off, K)], sidx[b], isem[b])
        pltpu.async_copy(edge_hbm.at[1, pl.ds(off, K)], didx[b], dsem[b])

    def wait_idx(b):
        pltpu.make_async_copy(edge_hbm.at[0, pl.ds(0, K)], sidx[b],
                              isem[b]).wait()
        pltpu.make_async_copy(edge_hbm.at[1, pl.ds(0, K)], didx[b],
                              dsem[b]).wait()

    def start_gather(b):
        pltpu.async_copy(y_hbm.at[sidx[b]], rows[b], gsem[b])

    def wait_gather(b):
        pltpu.make_async_copy(y_hbm.at[sidx[b]], rows[b], gsem[b]).wait()

    def copy_didx(b):
        for t in range(K // 16):
            sdidx[b][pl.ds(t * 16, 16)] = didx[b][pl.ds(t * 16, 16)]

    def start_scatter(b):
        pltpu.async_copy(rows[b], acc.at[sdidx[b]], ssem[b], add=True)

    def wait_scatter(b):
        pltpu.make_async_copy(rows[b], acc.at[sdidx[b]], ssem[b]).wait()

    nck = CH_W + jnp.where(w < TAIL, 1, 0)

    start_idx(0, 0)
    start_idx(1, 1)
    for t in range(ROWS_S // K):
        pltpu.make_async_copy(rows0,
                              acc.at[pl.ds(s * ROWS_S + t * K, K), :],
                              gs0).wait()
    plsc.subcore_barrier()
    wait_idx(0)
    start_gather(0)

    def pair(p, carry):
        j0 = 2 * p
        wait_gather(0)

        @pl.when(p > 0)
        def _():
            wait_scatter(1)

        wait_idx(1)
        start_gather(1)
        copy_didx(0)
        start_scatter(0)

        @pl.when(j0 + 2 < nck)
        def _():
            start_idx(j0 + 2, 0)

        wait_gather(1)
        wait_scatter(0)

        @pl.when(j0 + 2 < nck)
        def _():
            wait_idx(0)
            start_gather(0)

        copy_didx(1)
        start_scatter(1)

        @pl.when(j0 + 3 < nck)
        def _():
            start_idx(j0 + 3, 1)

        return carry

    lax.fori_loop(0, CH_W // 2, pair, 0)

    wait_scatter(1)

    @pl.when(w < TAIL)
    def _():
        wait_gather(0)
        copy_didx(0)
        start_scatter(0)
        wait_scatter(0)

    plsc.subcore_barrier()
    pltpu.sync_copy(acc.at[pl.ds(s * ROWS_S, ROWS_S), :],
                    out_hbm.at[c, pl.ds(s * ROWS_S, ROWS_S), :])


_agg_call = pl.kernel(
    _agg_body,
    out_type=jax.ShapeDtypeStruct((2, NP, D), jnp.float32),
    mesh=plsc.VectorSubcoreMesh(core_axis_name="c", subcore_axis_name="s"),
    scratch_types=[
        pltpu.VMEM_SHARED((NP, D), jnp.float32),
        pltpu.VMEM((K,), jnp.int32),
        pltpu.VMEM((K,), jnp.int32),
        pltpu.VMEM((K,), jnp.int32),
        pltpu.VMEM((K,), jnp.int32),
        pltpu.VMEM((K,), jnp.int32),
        pltpu.VMEM((K,), jnp.int32),
        pltpu.VMEM((K, D), jnp.float32),
        pltpu.VMEM((K, D), jnp.float32),
        pltpu.SemaphoreType.DMA,
        pltpu.SemaphoreType.DMA,
        pltpu.SemaphoreType.DMA,
        pltpu.SemaphoreType.DMA,
        pltpu.SemaphoreType.DMA,
        pltpu.SemaphoreType.DMA,
        pltpu.SemaphoreType.DMA,
        pltpu.SemaphoreType.DMA,
    ],
)


def _tc1b_body(degp, x_ref, w_ref, y_ref):
    deg = degp[0, :] + degp[1, :] + 1.0
    dinv = lax.rsqrt(deg)
    xw = jnp.dot(x_ref[...], w_ref[...], precision=_PREC,
                 preferred_element_type=jnp.float32)
    y_ref[...] = xw * dinv[:, None]


def _tc1b(degp, xp, W1):
    return pl.pallas_call(
        _tc1b_body,
        grid=(NB,),
        in_specs=[
            pl.BlockSpec((2, BN), lambda i: (0, i)),
            pl.BlockSpec((BN, D), lambda i: (i, 0)),
            pl.BlockSpec((D, H), lambda i: (0, 0)),
        ],
        out_specs=pl.BlockSpec((BN, H), lambda i: (i, 0)),
        out_shape=jax.ShapeDtypeStruct((NP, H), jnp.float32),
    )(degp, xp, W1)


def _tc2_body(degp, aggp, y_ref, b_ref, w_ref, out_ref):
    deg = degp[0, :] + degp[1, :] + 1.0
    dinv = lax.rsqrt(deg)
    agg = aggp[0] + aggp[1] + y_ref[...]
    h = jnp.maximum(agg * dinv[:, None] + b_ref[...], 0.0)
    hw = jnp.dot(h, w_ref[...], precision=_PREC,
                 preferred_element_type=jnp.float32)
    out_ref[...] = hw * dinv[:, None]


def _tc2(degp, aggp, y, b, W):
    return pl.pallas_call(
        _tc2_body,
        grid=(NB,),
        in_specs=[
            pl.BlockSpec((2, BN), lambda i: (0, i)),
            pl.BlockSpec((2, BN, D), lambda i: (0, i, 0)),
            pl.BlockSpec((BN, D), lambda i: (i, 0)),
            pl.BlockSpec((1, H), lambda i: (0, 0)),
            pl.BlockSpec((H, H), lambda i: (0, 0)),
        ],
        out_specs=pl.BlockSpec((BN, H), lambda i: (i, 0)),
        out_shape=jax.ShapeDtypeStruct((NP, H), jnp.float32),
    )(degp, aggp, y, b, W)


def _tc3_body(degp, aggp, y_ref, b_ref, wa_ref, ba_ref, wg_ref, bg_ref,
              batch_ref, wl_ref, bl_ref, out_ref,
              h2s, gates, m_acc, num_acc, den_acc):
    t = pl.program_id(0)
    i = pl.program_id(1)
    b = batch_ref[0, :]
    eq = b[:, None] == lax.broadcasted_iota(jnp.int32, (BN, G), 1)

    @pl.when(t == 0)
    def _():
        @pl.when(i == 0)
        def _():
            m_acc[...] = jnp.full((1, G), -jnp.inf, jnp.float32)

        deg = degp[0, :] + degp[1, :] + 1.0
        dinv = lax.rsqrt(deg)
        agg = aggp[0] + aggp[1] + y_ref[...]
        h2 = jnp.maximum(agg * dinv[:, None] + b_ref[...], 0.0)
        h2s[pl.ds(i * BN, BN), :] = h2
        a = jnp.maximum(jnp.dot(h2, wa_ref[...], precision=_PREC,
                                preferred_element_type=jnp.float32)
                        + ba_ref[...], 0.0)
        g = jnp.sum(a * wg_ref[...], axis=1) + bg_ref[0, 0]
        gates[0, pl.ds(i * BN, BN)] = g
        contrib = jnp.where(eq, g[:, None], -jnp.inf)
        m_acc[...] = jnp.maximum(m_acc[...], jnp.max(contrib, axis=0)[None, :])

    @pl.when(t == 1)
    def _():
        @pl.when(i == 0)
        def _():
            num_acc[...] = jnp.zeros((G, D), jnp.float32)
            den_acc[...] = jnp.zeros((G, D), jnp.float32)

        g = gates[0, pl.ds(i * BN, BN)]
        mm = m_acc[0, :]
        mm = jnp.where(mm == -jnp.inf, 0.0, mm)
        eqf = eq.astype(jnp.float32)
        mb = jnp.sum(eqf * mm[None, :], axis=1)
        e = jnp.where(b >= 0, jnp.exp(g - mb), 0.0)
        ew = eqf * e[:, None]
        dn = (((0,), (0,)), ((), ()))
        num_acc[...] += lax.dot_general(ew, h2s[pl.ds(i * BN, BN), :],
                                        dimension_numbers=dn, precision=_PREC,
                                        preferred_element_type=jnp.float32)
        den_acc[...] += lax.dot_general(ew, jnp.ones((BN, D), jnp.float32),
                                        dimension_numbers=dn, precision=_PREC,
                                        preferred_element_type=jnp.float32)

        @pl.when(i == NB - 1)
        def _():
            pooled = num_acc[...] / jnp.maximum(den_acc[...], 1e-16)
            out_ref[...] = jnp.dot(pooled, wl_ref[...], precision=_PREC,
                                   preferred_element_type=jnp.float32) \
                + bl_ref[...]


def _tc3(degp, aggp, y2, b2, Wa, ba, Wg, bg, batchp, Wl, bl):
    zero = lambda t, i: (0, 0)
    row = lambda t, i: (i, 0)
    rowt = lambda t, i: (0, i)
    return pl.pallas_call(
        _tc3_body,
        grid=(2, NB),
        in_specs=[
            pl.BlockSpec((2, BN), rowt),
            pl.BlockSpec((2, BN, D), lambda t, i: (0, i, 0)),
            pl.BlockSpec((BN, D), row),
            pl.BlockSpec((1, H), zero),
            pl.BlockSpec((H, H), zero),
            pl.BlockSpec((1, H), zero),
            pl.BlockSpec((1, H), zero),
            pl.BlockSpec((1, 1), zero),
            pl.BlockSpec((1, BN), rowt),
            pl.BlockSpec((H, C), zero),
            pl.BlockSpec((1, C), zero),
        ],
        out_specs=pl.BlockSpec((G, C), zero),
        out_shape=jax.ShapeDtypeStruct((G, C), jnp.float32),
        scratch_shapes=[pltpu.VMEM((NP, D), jnp.float32),
                        pltpu.VMEM((1, NP), jnp.float32),
                        pltpu.VMEM((1, G), jnp.float32),
                        pltpu.VMEM((G, D), jnp.float32),
                        pltpu.VMEM((G, D), jnp.float32)],
    )(degp, aggp, y2, b2, Wa, ba, Wg, bg, batchp, Wl, bl)


@jax.jit
def kernel(x, edge_index, batch, W1, b1, W2, b2, Wa, ba, Wg, bg, Wl, bl):
    xp = jnp.pad(x, ((0, NP - N), (0, 0)))
    batchp = jnp.pad(batch, (0, NP - N), constant_values=-1).reshape(1, NP)

    degp = _deg_call(edge_index)
    y1 = _tc1b(degp, xp, W1)
    aggp1 = _agg_call(y1, edge_index)
    y2 = _tc2(degp, aggp1, y1, b1.reshape(1, H), W2)
    aggp2 = _agg_call(y2, edge_index)
    out = _tc3(degp, aggp2, y2, b2.reshape(1, H), Wa, ba.reshape(1, H),
               Wg.reshape(1, H), bg.reshape(1, 1), batchp, Wl,
               bl.reshape(1, C))
    return out


# speedup vs baseline: 1.0124x; 1.0012x over previous
import jax
import jax.numpy as jnp
from jax import lax
from jax.experimental import pallas as pl
from jax.experimental.pallas import tpu as pltpu
from jax.experimental.pallas import tpu_sc as plsc

N = 10000
E = 320000
D = 128
H = 128
C = 10
G = 128

NP = 10240
NW = 32
ROWS_S = NP // 16
K = 128
NCHUNK = E // K
CH_W = NCHUNK // NW
TAIL = NCHUNK - CH_W * NW

BN = 1024
NB = NP // BN

_PREC = lax.Precision.HIGHEST


def _fill_f32(ref, n, val):
    def body(i, c):
        ref[pl.ds(i * 16, 16)] = jnp.full((16,), val, jnp.float32)
        return c
    lax.fori_loop(0, n // 16, body, 0)


def _fill2d_f32(ref, rows, val):
    def body(k, c):
        ref[k // 8, pl.ds((k % 8) * 16, 16)] = jnp.full((16,), val, jnp.float32)
        return c
    lax.fori_loop(0, rows * 8, body, 0)


def _deg_body(edge_hbm, out_hbm, acc, ones_v, zv, idx0, idx1, sx0, sx1,
              i0, i1, s0, s1):
    c = lax.axis_index("c")
    s = lax.axis_index("s")
    w = c * 16 + s
    idx = (idx0, idx1)
    sidx = (sx0, sx1)
    isem = (i0, i1)
    ssem = (s0, s1)
    _fill_f32(ones_v, K, 1.0)
    _fill_f32(zv, ROWS_S, 0.0)
    nck = CH_W + jnp.where(w < TAIL, 1, 0)

    def start_idx(j, b):
        off = (j * NW + w) * K
        pltpu.async_copy(edge_hbm.at[1, pl.ds(off, K)], idx[b], isem[b])

    def wait_idx(b):
        pltpu.make_async_copy(edge_hbm.at[1, pl.ds(0, K)], idx[b],
                              isem[b]).wait()

    def copy_idx(b):
        for t in range(K // 16):
            sidx[b][pl.ds(t * 16, 16)] = idx[b][pl.ds(t * 16, 16)]

    def start_scatter(b):
        pltpu.async_copy(ones_v, acc.at[sidx[b]], ssem[b], add=True)

    def wait_scatter(b):
        pltpu.make_async_copy(ones_v, acc.at[sidx[b]], ssem[b]).wait()

    start_idx(0, 0)
    start_idx(1, 1)
    pltpu.sync_copy(zv, acc.at[pl.ds(s * ROWS_S, ROWS_S)])
    plsc.subcore_barrier()

    def pair(p, carry):
        j0 = 2 * p
        wait_idx(0)

        @pl.when(p > 0)
        def _():
            wait_scatter(0)

        copy_idx(0)
        start_scatter(0)

        @pl.when(j0 + 2 < nck)
        def _():
            start_idx(j0 + 2, 0)

        wait_idx(1)

        @pl.when(p > 0)
        def _():
            wait_scatter(1)

        copy_idx(1)
        start_scatter(1)

        @pl.when(j0 + 3 < nck)
        def _():
            start_idx(j0 + 3, 1)

        return carry

    lax.fori_loop(0, CH_W // 2, pair, 0)
    wait_scatter(0)
    wait_scatter(1)

    @pl.when(w < TAIL)
    def _():
        wait_idx(0)
        copy_idx(0)
        start_scatter(0)
        wait_scatter(0)

    plsc.subcore_barrier()
    pltpu.sync_copy(acc.at[pl.ds(s * ROWS_S, ROWS_S)],
                    out_hbm.at[c, pl.ds(s * ROWS_S, ROWS_S)])


_deg_call = pl.kernel(
    _deg_body,
    out_type=jax.ShapeDtypeStruct((2, NP), jnp.float32),
    mesh=plsc.VectorSubcoreMesh(core_axis_name="c", subcore_axis_name="s"),
    scratch_types=[
        pltpu.VMEM_SHARED((NP,), jnp.float32),
        pltpu.VMEM((K,), jnp.float32),
        pltpu.VMEM((ROWS_S,), jnp.float32),
        pltpu.VMEM((K,), jnp.int32),
        pltpu.VMEM((K,), jnp.int32),
        pltpu.VMEM((K,), jnp.int32),
        pltpu.VMEM((K,), jnp.int32),
        pltpu.SemaphoreType.DMA,
        pltpu.SemaphoreType.DMA,
        pltpu.SemaphoreType.DMA,
        pltpu.SemaphoreType.DMA,
    ],
)


def _agg_body(y_hbm, edge_hbm, out_hbm, acc,
              sidx0, sidx1, didx0, didx1, sdidx0, sdidx1, rows0, rows1,
              is0, is1, id0, id1, gs0, gs1, ss0, ss1):
    c = lax.axis_index("c")
    s = lax.axis_index("s")
    w = c * 16 + s
    sidx = (sidx0, sidx1)
    didx = (didx0, didx1)
    sdidx = (sdidx0, sdidx1)
    rows = (rows0, rows1)
    isem = (is0, is1)
    dsem = (id0, id1)
    gsem = (gs0, gs1)
    ssem = (ss0, ss1)

    _fill2d_f32(rows0, K, 0.0)
    for t in range(ROWS_S // K):
        pltpu.async_copy(rows0, acc.at[pl.ds(s * ROWS_S + t * K, K), :], gs0)

    def start_idx(j, b):
        off = (j * NW + w) * K
        pltpu.async_copy(edge_hbm.at[0, pl.ds(off, K)], sidx[b], isem[b])
        pltpu.async_copy(edge_hbm.at[1, pl.ds(off, K)], didx[b], dsem[b])

    def wait_idx(b):
        pltpu.make_async_copy(edge_hbm.at[0, pl.ds(0, K)], sidx[b],
                              isem[b]).wait()
        pltpu.make_async_copy(edge_hbm.at[1, pl.ds(0, K)], didx[b],
                              dsem[b]).wait()

    def start_gather(b):
        pltpu.async_copy(y_hbm.at[sidx[b]], rows[b], gsem[b])

    def wait_gather(b):
        pltpu.make_async_copy(y_hbm.at[sidx[b]], rows[b], gsem[b]).wait()

    def copy_didx(b):
        for t in range(K // 16):
            sdidx[b][pl.ds(t * 16, 16)] = didx[b][pl.ds(t * 16, 16)]

    def start_scatter(b):
        pltpu.async_copy(rows[b], acc.at[sdidx[b]], ssem[b], add=True)

    def wait_scatter(b):
        pltpu.make_async_copy(rows[b], acc.at[sdidx[b]], ssem[b]).wait()

    nck = CH_W + jnp.where(w < TAIL, 1, 0)

    start_idx(0, 0)
    start_idx(1, 1)
    for t in range(ROWS_S // K):
        pltpu.make_async_copy(rows0,
                              acc.at[pl.ds(s * ROWS_S + t * K, K), :],
                              gs0).wait()
    plsc.subcore_barrier()
    wait_idx(0)
    start_gather(0)

    def pair(p, carry):
        j0 = 2 * p
        wait_gather(0)

        @pl.when(p > 0)
        def _():
            wait_scatter(1)

        wait_idx(1)
        start_gather(1)
        copy_didx(0)
        start_scatter(0)

        @pl.when(j0 + 2 < nck)
        def _():
            start_idx(j0 + 2, 0)

        wait_gather(1)
        wait_scatter(0)

        @pl.when(j0 + 2 < nck)
        def _():
            wait_idx(0)
            start_gather(0)

        copy_didx(1)
        start_scatter(1)

        @pl.when(j0 + 3 < nck)
        def _():
            start_idx(j0 + 3, 1)

        return carry

    lax.fori_loop(0, CH_W // 2, pair, 0)

    wait_scatter(1)

    @pl.when(w < TAIL)
    def _():
        wait_gather(0)
        copy_didx(0)
        start_scatter(0)
        wait_scatter(0)

    plsc.subcore_barrier()
    pltpu.sync_copy(acc.at[pl.ds(s * ROWS_S, ROWS_S), :],
                    out_hbm.at[c, pl.ds(s * ROWS_S, ROWS_S), :])


_agg_call = pl.kernel(
    _agg_body,
    out_type=jax.ShapeDtypeStruct((2, NP, D), jnp.float32),
    mesh=plsc.VectorSubcoreMesh(core_axis_name="c", subcore_axis_name="s"),
    scratch_types=[
        pltpu.VMEM_SHARED((NP, D), jnp.float32),
        pltpu.VMEM((K,), jnp.int32),
        pltpu.VMEM((K,), jnp.int32),
        pltpu.VMEM((K,), jnp.int32),
        pltpu.VMEM((K,), jnp.int32),
        pltpu.VMEM((K,), jnp.int32),
        pltpu.VMEM((K,), jnp.int32),
        pltpu.VMEM((K, D), jnp.float32),
        pltpu.VMEM((K, D), jnp.float32),
        pltpu.SemaphoreType.DMA,
        pltpu.SemaphoreType.DMA,
        pltpu.SemaphoreType.DMA,
        pltpu.SemaphoreType.DMA,
        pltpu.SemaphoreType.DMA,
        pltpu.SemaphoreType.DMA,
        pltpu.SemaphoreType.DMA,
        pltpu.SemaphoreType.DMA,
    ],
)


def _tc1b_body(degp, x_ref, w_ref, y_ref):
    deg = degp[0, :] + degp[1, :] + 1.0
    dinv = lax.rsqrt(deg)
    xw = jnp.dot(x_ref[...], w_ref[...], precision=_PREC,
                 preferred_element_type=jnp.float32)
    y_ref[...] = xw * dinv[:, None]


def _tc1b(degp, xp, W1):
    return pl.pallas_call(
        _tc1b_body,
        grid=(NB,),
        in_specs=[
            pl.BlockSpec((2, BN), lambda i: (0, i)),
            pl.BlockSpec((BN, D), lambda i: (i, 0)),
            pl.BlockSpec((D, H), lambda i: (0, 0)),
        ],
        out_specs=pl.BlockSpec((BN, H), lambda i: (i, 0)),
        out_shape=jax.ShapeDtypeStruct((NP, H), jnp.float32),
    )(degp, xp, W1)


def _tc2_body(degp, aggp, y_ref, b_ref, w_ref, out_ref):
    deg = degp[0, :] + degp[1, :] + 1.0
    dinv = lax.rsqrt(deg)
    agg = aggp[0] + aggp[1] + y_ref[...]
    h = jnp.maximum(agg * dinv[:, None] + b_ref[...], 0.0)
    hw = jnp.dot(h, w_ref[...], precision=_PREC,
                 preferred_element_type=jnp.float32)
    out_ref[...] = hw * dinv[:, None]


def _tc2(degp, aggp, y, b, W):
    return pl.pallas_call(
        _tc2_body,
        grid=(NB,),
        in_specs=[
            pl.BlockSpec((2, BN), lambda i: (0, i)),
            pl.BlockSpec((2, BN, D), lambda i: (0, i, 0)),
            pl.BlockSpec((BN, D), lambda i: (i, 0)),
            pl.BlockSpec((1, H), lambda i: (0, 0)),
            pl.BlockSpec((H, H), lambda i: (0, 0)),
        ],
        out_specs=pl.BlockSpec((BN, H), lambda i: (i, 0)),
        out_shape=jax.ShapeDtypeStruct((NP, H), jnp.float32),
    )(degp, aggp, y, b, W)


def _tc3_body(degp, aggp, y_ref, b_ref, wa_ref, ba_ref, wg_ref, bg_ref,
              batch_ref, wl_ref, bl_ref, out_ref,
              h2s, gates, m_acc, num_acc, den_acc):
    t = pl.program_id(0)
    i = pl.program_id(1)
    b = batch_ref[0, :]
    eq = b[:, None] == lax.broadcasted_iota(jnp.int32, (BN, G), 1)

    @pl.when(t == 0)
    def _():
        @pl.when(i == 0)
        def _():
            m_acc[...] = jnp.full((1, G), -jnp.inf, jnp.float32)

        deg = degp[0, :] + degp[1, :] + 1.0
        dinv = lax.rsqrt(deg)
        agg = aggp[0] + aggp[1] + y_ref[...]
        h2 = jnp.maximum(agg * dinv[:, None] + b_ref[...], 0.0)
        h2s[pl.ds(i * BN, BN), :] = h2
        a = jnp.maximum(jnp.dot(h2, wa_ref[...], precision=_PREC,
                                preferred_element_type=jnp.float32)
                        + ba_ref[...], 0.0)
        g = jnp.sum(a * wg_ref[...], axis=1) + bg_ref[0, 0]
        gates[0, pl.ds(i * BN, BN)] = g
        contrib = jnp.where(eq, g[:, None], -jnp.inf)
        m_acc[...] = jnp.maximum(m_acc[...], jnp.max(contrib, axis=0)[None, :])

    @pl.when(t == 1)
    def _():
        @pl.when(i == 0)
        def _():
            num_acc[...] = jnp.zeros((G, D), jnp.float32)
            den_acc[...] = jnp.zeros((G, D), jnp.float32)

        g = gates[0, pl.ds(i * BN, BN)]
        mm = m_acc[0, :]
        mm = jnp.where(mm == -jnp.inf, 0.0, mm)
        eqf = eq.astype(jnp.float32)
        mb = jnp.sum(eqf * mm[None, :], axis=1)
        e = jnp.where(b >= 0, jnp.exp(g - mb), 0.0)
        ew = eqf * e[:, None]
        dn = (((0,), (0,)), ((), ()))
        num_acc[...] += lax.dot_general(ew, h2s[pl.ds(i * BN, BN), :],
                                        dimension_numbers=dn, precision=_PREC,
                                        preferred_element_type=jnp.float32)
        den_acc[...] += lax.dot_general(ew, jnp.ones((BN, D), jnp.float32),
                                        dimension_numbers=dn, precision=_PREC,
                                        preferred_element_type=jnp.float32)

        @pl.when(i == NB - 1)
        def _():
            pooled = num_acc[...] / jnp.maximum(den_acc[...], 1e-16)
            out_ref[...] = jnp.dot(pooled, wl_ref[...], precision=_PREC,
                                   preferred_element_type=jnp.float32) \
                + bl_ref[...]


def _tc3(degp, aggp, y2, b2, Wa, ba, Wg, bg, batchp, Wl, bl):
    zero = lambda t, i: (0, 0)
    row = lambda t, i: (i, 0)
    rowt = lambda t, i: (0, i)
    return pl.pallas_call(
        _tc3_body,
        grid=(2, NB),
        in_specs=[
            pl.BlockSpec((2, BN), rowt),
            pl.BlockSpec((2, BN, D), lambda t, i: (0, i, 0)),
            pl.BlockSpec((BN, D), row),
            pl.BlockSpec((1, H), zero),
            pl.BlockSpec((H, H), zero),
            pl.BlockSpec((1, H), zero),
            pl.BlockSpec((1, H), zero),
            pl.BlockSpec((1, 1), zero),
            pl.BlockSpec((1, BN), rowt),
            pl.BlockSpec((H, C), zero),
            pl.BlockSpec((1, C), zero),
        ],
        out_specs=pl.BlockSpec((G, C), zero),
        out_shape=jax.ShapeDtypeStruct((G, C), jnp.float32),
        scratch_shapes=[pltpu.VMEM((NP, D), jnp.float32),
                        pltpu.VMEM((1, NP), jnp.float32),
                        pltpu.VMEM((1, G), jnp.float32),
                        pltpu.VMEM((G, D), jnp.float32),
                        pltpu.VMEM((G, D), jnp.float32)],
    )(degp, aggp, y2, b2, Wa, ba, Wg, bg, batchp, Wl, bl)


@jax.jit
def kernel(x, edge_index, batch, W1, b1, W2, b2, Wa, ba, Wg, bg, Wl, bl):
    xp = jnp.pad(x, ((0, NP - N), (0, 0)))
    batchp = jnp.pad(batch, (0, NP - N), constant_values=-1).reshape(1, NP)

    degp = _deg_call(edge_index)
    y1 = _tc1b(degp, xp, W1)
    aggp1 = _agg_call(y1, edge_index)
    y2 = _tc2(degp, aggp1, y1, b1.reshape(1, H), W2)
    aggp2 = _agg_call(y2, edge_index)
    out = _tc3(degp, aggp2, y2, b2.reshape(1, H), Wa, ba.reshape(1, H),
               Wg.reshape(1, H), bg.reshape(1, 1), batchp, Wl,
               bl.reshape(1, C))
    return out
